# Initial kernel scaffold; baseline (speedup 1.0000x reference)
import jax
import jax.numpy as jnp
from jax.experimental import pallas as pl


def kernel(x, y, batch_c, a):
    raise NotImplementedError("write your pallas kernel here")


# speedup vs baseline: 3.2256x; 3.2256x over previous
import functools

import jax
import jax.numpy as jnp
from jax import lax
from jax.experimental import pallas as pl
from jax.experimental.pallas import tpu as pltpu
from jax.experimental.pallas import tpu_sc as plsc

B = 16384
F = 128
N_TAB = 1000
A_PAD = 1024

NC = 2
NS = 16
NW = NC * NS
ROWS_PER_W = B // NW
CHUNK = 256
NCHUNK = ROWS_PER_W // CHUNK
LANES = 16
GROUPS = CHUNK // LANES


def _sc_body(x_hbm, y_hbm, c_hbm, a_hbm, out_hbm,
             idx_a, idx_b, a_tab, xbuf, cbuf, sem_x, sem_c):
    wid = lax.axis_index("s") * NC + lax.axis_index("c")
    base = wid * ROWS_PER_W

    pltpu.sync_copy(a_hbm, a_tab)

    for j, idx in enumerate((idx_a, idx_b)):
        cb = base + j * CHUNK
        pltpu.sync_copy(y_hbm.at[pl.ds(cb, CHUNK)], idx)
        cp_x = pltpu.async_copy(x_hbm.at[pl.ds(cb, CHUNK), :], xbuf, sem_x)
        cp_c = pltpu.async_copy(c_hbm.at[idx], cbuf, sem_c)
        cp_x.wait()
        cp_c.wait()

        def group_body(g, carry):
            idxg = idx[pl.ds(g * LANES, LANES)]
            av = plsc.load_gather(a_tab, [idxg])
            for r in range(LANES):
                s = av[r]
                row = g * LANES + r
                for k in range(F // LANES):
                    sl = pl.ds(k * LANES, LANES)
                    cbuf[row, sl] = xbuf[row, sl] * s + cbuf[row, sl]
            return carry

        lax.fori_loop(0, GROUPS, group_body, 0)
        pltpu.sync_copy(cbuf, out_hbm.at[pl.ds(cb, CHUNK), :])


@jax.jit
def _run(x, y, c_pad, a_flat):
    mesh = plsc.VectorSubcoreMesh(core_axis_name="c", subcore_axis_name="s")
    fn = functools.partial(
        pl.kernel,
        out_type=jax.ShapeDtypeStruct((B, F), jnp.float32),
        mesh=mesh,
        scratch_types=[
            pltpu.VMEM((CHUNK,), jnp.int32),
            pltpu.VMEM((CHUNK,), jnp.int32),
            pltpu.VMEM((A_PAD,), jnp.float32),
            pltpu.VMEM((CHUNK, F), jnp.float32),
            pltpu.VMEM((CHUNK, F), jnp.float32),
            pltpu.SemaphoreType.DMA,
            pltpu.SemaphoreType.DMA,
        ],
        compiler_params=pltpu.CompilerParams(needs_layout_passes=False),
    )(_sc_body)
    return fn(x, y, c_pad, a_flat)


def kernel(x, y, batch_c, a):
    a_flat = jnp.pad(a.reshape(-1), (0, A_PAD - N_TAB))
    c_pad = jnp.pad(batch_c, ((0, A_PAD - N_TAB), (0, 0)))
    return _run(x, y.astype(jnp.int32), c_pad, a_flat)
